# baseline (speedup 1.0000x reference)
import functools

import jax
import jax.numpy as jnp
from jax import lax
from jax.experimental import pallas as pl
from jax.experimental.pallas import tpu as pltpu
from jax.experimental.pallas import tpu_sc as plsc
from jax.experimental.layout import Layout, with_layout_constraint

VOCAB = 1000000
DIM = 64
BATCH = 16384
HIST = 200

_NC = 2
_NS = 16
_NW = _NC * _NS

_BT = BATCH * HIST
_BTILE = 128
_TCS = BATCH // _BTILE
_TC_PER_W = _TCS // _NW
_PAIRS = HIST // 2


def _make_gather():
    mesh = plsc.VectorSubcoreMesh(core_axis_name="c", subcore_axis_name="s")

    @functools.partial(
        pl.kernel,
        mesh=mesh,
        out_type=jax.ShapeDtypeStruct((HIST, DIM // 8, _TCS * 8 * 128),
                                      jnp.float32),
        scratch_types=[
            pltpu.VMEM((_BTILE, HIST), jnp.int32),
            pltpu.VMEM((_BTILE,), jnp.int32),
            pltpu.VMEM((_BTILE,), jnp.int32),
            pltpu.VMEM((_BTILE, DIM), jnp.float32),
            pltpu.VMEM((_BTILE, DIM), jnp.float32),
            pltpu.VMEM((DIM // 8, 8 * _BTILE), jnp.float32),
            pltpu.VMEM((DIM // 8, 8 * _BTILE), jnp.float32),
            pltpu.SemaphoreType.DMA,
            pltpu.SemaphoreType.DMA,
            pltpu.SemaphoreType.DMA,
            pltpu.SemaphoreType.DMA,
        ],
        compiler_params=pltpu.CompilerParams(use_tc_tiling_on_sc=False,
                                             needs_layout_passes=False),
    )
    def gather_kernel(idx_hbm, table_hbm, out_hbm,
                      xblk, idx0, idx1, g0, g1, t0, t1,
                      semg0, semg1, sems0, sems1):
        wid = lax.axis_index("s") * _NC + lax.axis_index("c")
        idx = (idx0, idx1)
        g = (g0, g1)
        t = (t0, t1)
        semg = (semg0, semg1)
        sems = (sems0, sems1)
        iota = lax.iota(jnp.int32, 16)

        def build_idx(l, b):
            lsplat = iota * 0 + l
            for q in range(8):
                bv = iota + q * 16
                v = plsc.load_gather(xblk, [bv, lsplat])
                idx[b][pl.ds(q * 16, 16)] = v

        def start_gather(l, b):
            build_idx(l, b)
            pltpu.async_copy(table_hbm.at[idx[b]], g[b], semg[b])

        def wait_gather(b):
            pltpu.make_async_copy(table_hbm.at[idx[b]], g[b], semg[b]).wait()

        prow = []
        pcol = []
        for q in range(DIM // 16):
            d = q * 16 + iota
            prow.append(lax.shift_right_logical(d, 2 + 1))
            pcol.append((d & 7) * 128)

        dcol = []
        for q in range(DIM // 16):
            dcol.append(q * 16 + iota)

        def transpose(b):
            def jblock(jj, carry):
                j0 = jj * 16
                for u in range(16):
                    jsplat = iota * 0 + (j0 + u)
                    for q in range(DIM // 16):
                        vals = plsc.load_gather(g[b], [jsplat, dcol[q]])
                        plsc.store_scatter(t[b],
                                           [prow[q], pcol[q] + (j0 + u)],
                                           vals)
                return carry

            lax.fori_loop(0, _BTILE // 16, jblock, 0)

        def start_store(l, tc, b):
            pltpu.async_copy(t[b], out_hbm.at[l, :, pl.ds(tc * 1024, 1024)],
                             sems[b])

        def wait_store(l, tc, b):
            pltpu.make_async_copy(t[b],
                                  out_hbm.at[l, :, pl.ds(tc * 1024, 1024)],
                                  sems[b]).wait()

        for ti in range(_TC_PER_W):
            tc = wid * _TC_PER_W + ti
            pltpu.sync_copy(idx_hbm.at[pl.ds(tc * _BTILE, _BTILE)], xblk)
            start_gather(0, 0)
            start_gather(1, 1)

            def pair(p, carry):
                for b in range(2):
                    l = 2 * p + b
                    wait_gather(b)

                    @pl.when(p >= 1)
                    def _():
                        wait_store(l - 2, tc, b)

                    transpose(b)

                    @pl.when(p < _PAIRS - 1)
                    def _():
                        start_gather(l + 2, b)

                    start_store(l, tc, b)
                return carry

            lax.fori_loop(0, _PAIRS, pair, 0)

            wait_store(HIST - 2, tc, 0)
            wait_store(HIST - 1, tc, 1)

    return gather_kernel


_gather = _make_gather()


def kernel(x, table):
    out5 = _gather(x.astype(jnp.int32), table)
    o = out5.reshape(HIST, DIM // 8, _TCS, 8, _BTILE)
    o = with_layout_constraint(o, Layout((0, 1, 2, 3, 4),
                                         tiling=((8, 128),)))
    y = o.transpose(2, 4, 0, 1, 3)
    y = with_layout_constraint(y, Layout((2, 3, 0, 4, 1),
                                         tiling=((8, 128),)))
    return y.reshape(BATCH, HIST, DIM)


# speedup vs baseline: 1.6045x; 1.6045x over previous
import functools

import jax
import jax.numpy as jnp
from jax import lax
from jax.experimental import pallas as pl
from jax.experimental.pallas import tpu as pltpu
from jax.experimental.pallas import tpu_sc as plsc
from jax.experimental.layout import Layout, with_layout_constraint

VOCAB = 1000000
DIM = 64
BATCH = 16384
HIST = 200

_NC = 2
_NS = 16
_NW = _NC * _NS

_BT = BATCH * HIST
_ROWS_PER_W = BATCH // _NW
_R = 4
_CHUNK = _R * HIST
_STEPS = _ROWS_PER_W // _R
_PAIRS = _STEPS // 2


def _make_gather():
    mesh = plsc.VectorSubcoreMesh(core_axis_name="c", subcore_axis_name="s")

    @functools.partial(
        pl.kernel,
        mesh=mesh,
        out_type=jax.ShapeDtypeStruct((BATCH, HIST, DIM), jnp.float32),
        scratch_types=[
            pltpu.VMEM((_CHUNK,), jnp.int32),
            pltpu.VMEM((_CHUNK,), jnp.int32),
            pltpu.VMEM((_CHUNK, DIM), jnp.float32),
            pltpu.VMEM((_CHUNK, DIM), jnp.float32),
            pltpu.SemaphoreType.DMA,
            pltpu.SemaphoreType.DMA,
            pltpu.SemaphoreType.DMA,
            pltpu.SemaphoreType.DMA,
        ],
        compiler_params=pltpu.CompilerParams(use_tc_tiling_on_sc=False),
    )
    def gather_kernel(idx_hbm, table_hbm, out_hbm,
                      idx0, idx1, rows0, rows1,
                      semg0, semg1, sems0, sems1):
        wid = lax.axis_index("s") * _NC + lax.axis_index("c")
        row_base = wid * _ROWS_PER_W
        idx = (idx0, idx1)
        rows = (rows0, rows1)
        semg = (semg0, semg1)
        sems = (sems0, sems1)

        def issue_gather(c, b):
            off = (row_base + c * _R) * HIST
            pltpu.sync_copy(idx_hbm.at[pl.ds(off, _CHUNK)], idx[b])
            pltpu.async_copy(table_hbm.at[idx[b]], rows[b], semg[b])

        def issue_stores(c, b):
            r0 = row_base + c * _R
            for k in range(_R):
                pltpu.async_copy(rows[b].at[pl.ds(k * HIST, HIST)],
                                 out_hbm.at[r0 + k], sems[b])

        def wait_stores(c, b):
            r0 = row_base + c * _R
            for k in range(_R):
                pltpu.make_async_copy(rows[b].at[pl.ds(k * HIST, HIST)],
                                      out_hbm.at[r0 + k], sems[b]).wait()

        issue_gather(0, 0)
        issue_gather(1, 1)

        def pair(p, carry):
            for b in range(2):
                c = 2 * p + b
                pltpu.make_async_copy(table_hbm.at[idx[b]], rows[b],
                                      semg[b]).wait()
                issue_stores(c, b)

                @pl.when(p < _PAIRS - 1)
                def _():
                    wait_stores(c, b)
                    issue_gather(c + 2, b)

            return carry

        lax.fori_loop(0, _PAIRS, pair, 0)

        wait_stores(_STEPS - 2, 0)
        wait_stores(_STEPS - 1, 1)

    return gather_kernel


_gather = _make_gather()


def kernel(x, table):
    xf = with_layout_constraint(x.reshape(_BT).astype(jnp.int32),
                                Layout((0,), tiling=()))
    tf = with_layout_constraint(table, Layout((0, 1), tiling=()))
    out = _gather(xf, tf)
    out = with_layout_constraint(out, Layout((0, 1, 2), tiling=()))
    return out * 1.0
